# baseline (speedup 1.0000x reference)
import functools

import jax
import jax.numpy as jnp
from jax import lax
from jax.experimental import pallas as pl
from jax.experimental.pallas import tpu as pltpu
from jax.experimental.pallas import tpu_sc as plsc

N = 20000
PRE_NMS_TOPK = 6000
AFT_NMS_TOPK = 300
RPN_NMS_THR = 0.7

L = 16
NPAD = 20224
NB1 = NPAD // L
NB1_PAD = 1280
NB2 = NB1_PAD // L
KEEP_PAD = 336
NEG_INF = float("-inf")


def _decode_body(im_ref, a_ref, r_ref, o_ref):
    a = a_ref[...]
    r = r_ref[...]
    x1, y1, x2, y2 = a[0:1], a[1:2], a[2:3], a[3:4]
    w = x2 - x1 + 1.0
    h = y2 - y1 + 1.0
    cx = x1 + 0.5 * w
    cy = y1 + 0.5 * h
    pcx = r[0:1] * w + cx
    pcy = r[1:2] * h + cy
    pw = jnp.exp(r[2:3]) * w
    ph = jnp.exp(r[3:4]) * h
    rois = jnp.concatenate(
        [pcx - 0.5 * pw, pcy - 0.5 * ph, pcx + 0.5 * pw, pcy + 0.5 * ph], axis=0)
    col = lax.broadcasted_iota(jnp.int32, rois.shape, 1)
    even = (col % 2) == 0
    lim0 = im_ref[0] - 1.0
    lim1 = im_ref[1] - 1.0
    o_ref[...] = jnp.where(even, jnp.clip(rois, 0.0, lim0),
                           jnp.clip(rois, 0.0, lim1))


def _decode_rois(im_info, anchors_t, reg_t):
    return pl.pallas_call(
        _decode_body,
        out_shape=jax.ShapeDtypeStruct((4, N), jnp.float32),
        in_specs=[
            pl.BlockSpec(memory_space=pltpu.SMEM),
            pl.BlockSpec(memory_space=pltpu.VMEM),
            pl.BlockSpec(memory_space=pltpu.VMEM),
        ],
        out_specs=pl.BlockSpec(memory_space=pltpu.VMEM),
    )(im_info, anchors_t, reg_t)


def _splat_f(x):
    return jnp.full((L,), x, dtype=jnp.float32)


def _splat_i(x):
    return jnp.full((L,), x, dtype=jnp.int32)


def _sc_body(score_hbm, rois_hbm, out_hbm, sc_v, rois_v, bm1_v, bm2_v,
             kept_v, kx2p_v, ky2p_v, kathr_v, sem):
    iota = lax.iota(jnp.int32, L)
    big = jnp.int32(0x7FFFFFF)
    thr = jnp.float32(RPN_NMS_THR)
    thr1 = jnp.float32(1.0 + RPN_NMS_THR)
    bfly = [iota ^ s for s in (1, 2, 4, 8)]

    def _shuf(v, p):
        return v.at[p].get(mode="promise_in_bounds")

    def tree_max(v):
        for p in bfly:
            v = jnp.maximum(v, _shuf(v, p))
        return v

    def tree_min(v):
        for p in bfly:
            v = jnp.minimum(v, _shuf(v, p))
        return v

    @pl.when((lax.axis_index("c") == 0) & (lax.axis_index("s") == 0))
    def _tile0():
        rois_cp = pltpu.async_copy(rois_hbm, rois_v, sem)
        pltpu.sync_copy(score_hbm, sc_v)

        def bm1_chunk(k, _):
            m = _splat_f(NEG_INF)

            def inner(j, m):
                v = plsc.load_gather(sc_v, [(k * 256 + iota * L) + j])
                return jnp.maximum(m, v)

            m = lax.fori_loop(0, L, inner, m, unroll=4)
            plsc.store_scatter(bm1_v, [k * L + iota], m)
            return 0

        lax.fori_loop(0, NB1 // L, bm1_chunk, 0)
        bm1_v[pl.ds(NB1, L)] = _splat_f(NEG_INF)

        def bm2_one(t, _):
            v = plsc.load_gather(bm1_v, [t * L + iota])
            plsc.store_scatter(bm2_v, [_splat_i(t)], tree_max(v))
            return 0

        lax.fori_loop(0, NB2, bm2_one, 0)

        def kept_init(i, _):
            cols = i * L + iota
            plsc.store_scatter(kept_v, [_splat_i(0), cols], _splat_f(3e9))
            plsc.store_scatter(kept_v, [_splat_i(1), cols], _splat_f(3e9))
            plsc.store_scatter(kept_v, [_splat_i(2), cols], _splat_f(-3e9))
            plsc.store_scatter(kept_v, [_splat_i(3), cols], _splat_f(-3e9))
            plsc.store_scatter(kx2p_v, [cols], _splat_f(-3e9))
            plsc.store_scatter(ky2p_v, [cols], _splat_f(-3e9))
            plsc.store_scatter(kathr_v, [cols], _splat_f(1.0))
            return 0

        lax.fori_loop(0, KEEP_PAD // L, kept_init, 0)

        rois_cp.wait()

        def first_min_idx(vals, maxv, idxs):
            return jnp.min(jnp.where(vals == maxv, idxs, big))

        def draw():
            vs = [(bm2_v[pl.ds(t * L, L)], t * L + iota)
                  for t in range(NB2 // L)]
            while len(vs) > 1:
                nxt = []
                for i in range(0, len(vs) - 1, 2):
                    (va, ka), (vb, kb) = vs[i], vs[i + 1]
                    later = vb > va
                    nxt.append((jnp.where(later, vb, va),
                                jnp.where(later, kb, ka)))
                if len(vs) % 2:
                    nxt.append(vs[-1])
                vs = nxt
            bestv, bestk = vs[0]
            m2 = tree_max(bestv)
            k_star = tree_min(jnp.where(bestv == m2, bestk, big))
            v1 = plsc.load_gather(bm1_v, [k_star * L + iota])
            m1 = tree_max(v1)
            l1 = plsc.all_reduce_ffs(v1 == m1)
            b_star = k_star * L + l1
            v0 = plsc.load_gather(sc_v, [b_star * L + iota])
            m0 = tree_max(v0)
            l0 = plsc.all_reduce_ffs(v0 == m0)
            widx = b_star * L + l0

            plsc.store_scatter(sc_v, [widx], _splat_f(NEG_INF))
            v0n = jnp.where(iota == l0, NEG_INF, v0)
            bm1_new = tree_max(v0n)
            plsc.store_scatter(bm1_v, [b_star], bm1_new)
            v1n = jnp.where(iota == l1, bm1_new, v1)
            plsc.store_scatter(bm2_v, [k_star], tree_max(v1n))

            x1 = plsc.load_gather(rois_v, [_splat_i(0), widx])
            y1 = plsc.load_gather(rois_v, [_splat_i(1), widx])
            x2 = plsc.load_gather(rois_v, [_splat_i(2), widx])
            y2 = plsc.load_gather(rois_v, [_splat_i(3), widx])
            x2p = x2 + 1.0
            y2p = y2 + 1.0
            thr_area = thr * ((x2p - x1) * (y2p - y1))
            return (x1, y1, x2, y2, x2p, y2p, thr_area)

        def append(cand, slot):
            x1, y1, x2, y2, x2p, y2p, thr_area = cand
            csplat = _splat_i(slot)
            plsc.store_scatter(kept_v, [_splat_i(0), csplat], x1)
            plsc.store_scatter(kept_v, [_splat_i(1), csplat], y1)
            plsc.store_scatter(kept_v, [_splat_i(2), csplat], x2)
            plsc.store_scatter(kept_v, [_splat_i(3), csplat], y2)
            plsc.store_scatter(kx2p_v, [csplat], x2p)
            plsc.store_scatter(ky2p_v, [csplat], y2p)
            plsc.store_scatter(kathr_v, [csplat], thr_area)

        def cond(carry):
            count, picked = carry
            return (count < AFT_NMS_TOPK) & (picked < PRE_NMS_TOPK)

        def body(carry):
            count, picked = carry
            ca = draw()
            cb = draw()
            ax1, ay1, _, _, ax2p, ay2p, a_ta = ca
            bx1, by1, _, _, bx2p, by2p, b_ta = cb

            def iou_chunk(cols, accs):
                acca, accb = accs
                kx1 = plsc.load_gather(kept_v, [_splat_i(0), cols])
                ky1 = plsc.load_gather(kept_v, [_splat_i(1), cols])
                kx2p = plsc.load_gather(kx2p_v, [cols])
                ky2p = plsc.load_gather(ky2p_v, [cols])
                kathr = plsc.load_gather(kathr_v, [cols])
                iwa = jnp.maximum(0.0, jnp.minimum(ax2p, kx2p) - jnp.maximum(ax1, kx1))
                iha = jnp.maximum(0.0, jnp.minimum(ay2p, ky2p) - jnp.maximum(ay1, ky1))
                acca = acca | (thr1 * (iwa * iha) > a_ta + kathr)
                iwb = jnp.maximum(0.0, jnp.minimum(bx2p, kx2p) - jnp.maximum(bx1, kx1))
                ihb = jnp.maximum(0.0, jnp.minimum(by2p, ky2p) - jnp.maximum(by1, ky1))
                accb = accb | (thr1 * (iwb * ihb) > b_ta + kathr)
                return (acca, accb)

            def iou_pair(j, accs):
                cols = j * (2 * L) + iota
                return iou_chunk(cols + L, iou_chunk(cols, accs))

            npair = (count + (2 * L - 1)) // (2 * L)
            zero = jnp.zeros((L,), dtype=jnp.bool_)
            supp_a, supp_b = lax.fori_loop(0, npair, iou_pair, (zero, zero))
            keep_a = jnp.logical_not(jnp.any(supp_a))

            iw = jnp.maximum(0.0, jnp.minimum(ax2p, bx2p) - jnp.maximum(ax1, bx1))
            ih = jnp.maximum(0.0, jnp.minimum(ay2p, by2p) - jnp.maximum(ay1, by1))
            ab = jnp.any(thr1 * (iw * ih) > a_ta + b_ta)
            keep_b = jnp.logical_not(jnp.any(supp_b)) & ~(keep_a & ab)

            @pl.when(keep_a)
            def _():
                append(ca, count)

            cnt_a = count + keep_a.astype(jnp.int32)
            keep_b = keep_b & (cnt_a < AFT_NMS_TOPK)

            @pl.when(keep_b)
            def _():
                append(cb, cnt_a)

            return (cnt_a + keep_b.astype(jnp.int32), picked + 2)

        count, _ = lax.while_loop(cond, body, (jnp.int32(0), jnp.int32(0)))

        k0 = [plsc.load_gather(kept_v, [_splat_i(r), _splat_i(0)])
              for r in range(4)]

        def pad_chunk(i, _):
            cols = i * L + iota
            live = cols < count
            for r in range(4):
                cur = plsc.load_gather(kept_v, [_splat_i(r), cols])
                plsc.store_scatter(kept_v, [_splat_i(r), cols],
                                   jnp.where(live, cur, k0[r]))
            return 0

        lax.fori_loop(0, KEEP_PAD // L, pad_chunk, 0)

        pltpu.sync_copy(kept_v, out_hbm)


_sc_nms = functools.partial(
    pl.kernel,
    out_type=jax.ShapeDtypeStruct((4, KEEP_PAD), jnp.float32),
    mesh=plsc.VectorSubcoreMesh(core_axis_name="c", subcore_axis_name="s"),
    compiler_params=pltpu.CompilerParams(needs_layout_passes=False),
    scratch_types=[
        pltpu.VMEM((NPAD,), jnp.float32),
        pltpu.VMEM((4, N), jnp.float32),
        pltpu.VMEM((NB1_PAD,), jnp.float32),
        pltpu.VMEM((NB2,), jnp.float32),
        pltpu.VMEM((4, KEEP_PAD), jnp.float32),
        pltpu.VMEM((KEEP_PAD,), jnp.float32),
        pltpu.VMEM((KEEP_PAD,), jnp.float32),
        pltpu.VMEM((KEEP_PAD,), jnp.float32),
        pltpu.SemaphoreType.DMA,
    ],
)(_sc_body)


def kernel(score, reg_param, anchors, im_info):
    score_fg = jnp.concatenate(
        [score[0, :, 0], jnp.full((NPAD - N,), NEG_INF, dtype=jnp.float32)])
    rois_t = _decode_rois(im_info.astype(jnp.float32), anchors.T, reg_param.T)
    out = _sc_nms(score_fg, rois_t)
    return out[:, :AFT_NMS_TOPK].T


# speedup vs baseline: 1.0863x; 1.0863x over previous
import functools

import jax
import jax.numpy as jnp
from jax import lax
from jax.experimental import pallas as pl
from jax.experimental.pallas import tpu as pltpu
from jax.experimental.pallas import tpu_sc as plsc

N = 20000
PRE_NMS_TOPK = 6000
AFT_NMS_TOPK = 300
RPN_NMS_THR = 0.7

L = 16
NPAD = 20224
NB1 = NPAD // L
NB1_PAD = 1280
NB2 = NB1_PAD // L
KEEP_PAD = 336
NEG_INF = float("-inf")


def _decode_body(im_ref, a_ref, r_ref, o_ref):
    a = a_ref[...]
    r = r_ref[...]
    x1, y1, x2, y2 = a[0:1], a[1:2], a[2:3], a[3:4]
    w = x2 - x1 + 1.0
    h = y2 - y1 + 1.0
    cx = x1 + 0.5 * w
    cy = y1 + 0.5 * h
    pcx = r[0:1] * w + cx
    pcy = r[1:2] * h + cy
    pw = jnp.exp(r[2:3]) * w
    ph = jnp.exp(r[3:4]) * h
    rois = jnp.concatenate(
        [pcx - 0.5 * pw, pcy - 0.5 * ph, pcx + 0.5 * pw, pcy + 0.5 * ph], axis=0)
    col = lax.broadcasted_iota(jnp.int32, rois.shape, 1)
    even = (col % 2) == 0
    lim0 = im_ref[0] - 1.0
    lim1 = im_ref[1] - 1.0
    o_ref[...] = jnp.where(even, jnp.clip(rois, 0.0, lim0),
                           jnp.clip(rois, 0.0, lim1))


def _decode_rois(im_info, anchors_t, reg_t):
    return pl.pallas_call(
        _decode_body,
        out_shape=jax.ShapeDtypeStruct((4, N), jnp.float32),
        in_specs=[
            pl.BlockSpec(memory_space=pltpu.SMEM),
            pl.BlockSpec(memory_space=pltpu.VMEM),
            pl.BlockSpec(memory_space=pltpu.VMEM),
        ],
        out_specs=pl.BlockSpec(memory_space=pltpu.VMEM),
    )(im_info, anchors_t, reg_t)


def _splat_f(x):
    return jnp.full((L,), x, dtype=jnp.float32)


def _splat_i(x):
    return jnp.full((L,), x, dtype=jnp.int32)


def _sc_body(score_hbm, rois_hbm, out_hbm, sc_v, rois_v, bm1_v, bm2_v,
             kept_v, kx2p_v, ky2p_v, kathr_v, sem):
    iota = lax.iota(jnp.int32, L)
    big = jnp.int32(0x7FFFFFF)
    thr = jnp.float32(RPN_NMS_THR)
    thr1 = jnp.float32(1.0 + RPN_NMS_THR)

    @pl.when((lax.axis_index("c") == 0) & (lax.axis_index("s") == 0))
    def _tile0():
        rois_cp = pltpu.async_copy(rois_hbm, rois_v, sem)
        pltpu.sync_copy(score_hbm, sc_v)

        def bm1_chunk(k, _):
            m = _splat_f(NEG_INF)

            def inner(j, m):
                v = plsc.load_gather(sc_v, [(k * 256 + iota * L) + j])
                return jnp.maximum(m, v)

            m = lax.fori_loop(0, L, inner, m, unroll=4)
            plsc.store_scatter(bm1_v, [k * L + iota], m)
            return 0

        lax.fori_loop(0, NB1 // L, bm1_chunk, 0)
        bm1_v[pl.ds(NB1, L)] = _splat_f(NEG_INF)

        def bm2_one(t, _):
            v = plsc.load_gather(bm1_v, [t * L + iota])
            plsc.store_scatter(bm2_v, [_splat_i(t)], _splat_f(jnp.max(v)))
            return 0

        lax.fori_loop(0, NB2, bm2_one, 0)

        def kept_init(i, _):
            cols = i * L + iota
            plsc.store_scatter(kept_v, [_splat_i(0), cols], _splat_f(3e9))
            plsc.store_scatter(kept_v, [_splat_i(1), cols], _splat_f(3e9))
            plsc.store_scatter(kept_v, [_splat_i(2), cols], _splat_f(-3e9))
            plsc.store_scatter(kept_v, [_splat_i(3), cols], _splat_f(-3e9))
            plsc.store_scatter(kx2p_v, [cols], _splat_f(-3e9))
            plsc.store_scatter(ky2p_v, [cols], _splat_f(-3e9))
            plsc.store_scatter(kathr_v, [cols], _splat_f(1.0))
            return 0

        lax.fori_loop(0, KEEP_PAD // L, kept_init, 0)

        rois_cp.wait()

        def first_min_idx(vals, maxv, idxs):
            return jnp.min(jnp.where(vals == maxv, idxs, big))

        def draw():
            vs = [(bm2_v[pl.ds(t * L, L)], t * L + iota)
                  for t in range(NB2 // L)]
            while len(vs) > 1:
                nxt = []
                for i in range(0, len(vs) - 1, 2):
                    (va, ka), (vb, kb) = vs[i], vs[i + 1]
                    later = vb > va
                    nxt.append((jnp.where(later, vb, va),
                                jnp.where(later, kb, ka)))
                if len(vs) % 2:
                    nxt.append(vs[-1])
                vs = nxt
            bestv, bestk = vs[0]
            m2 = jnp.max(bestv)
            k_star = first_min_idx(bestv, m2, bestk)
            v1 = plsc.load_gather(bm1_v, [k_star * L + iota])
            l1 = plsc.all_reduce_ffs(v1 == jnp.max(v1))
            b_star = k_star * L + l1
            v0 = plsc.load_gather(sc_v, [b_star * L + iota])
            l0 = plsc.all_reduce_ffs(v0 == jnp.max(v0))
            widx = b_star * L + l0

            plsc.store_scatter(sc_v, [_splat_i(widx)], _splat_f(NEG_INF))
            v0n = jnp.where(iota == l0, NEG_INF, v0)
            bm1_new = jnp.max(v0n)
            plsc.store_scatter(bm1_v, [_splat_i(b_star)], _splat_f(bm1_new))
            v1n = jnp.where(iota == l1, bm1_new, v1)
            plsc.store_scatter(bm2_v, [_splat_i(k_star)], _splat_f(jnp.max(v1n)))

            wsplat = _splat_i(widx)
            x1 = plsc.load_gather(rois_v, [_splat_i(0), wsplat])
            y1 = plsc.load_gather(rois_v, [_splat_i(1), wsplat])
            x2 = plsc.load_gather(rois_v, [_splat_i(2), wsplat])
            y2 = plsc.load_gather(rois_v, [_splat_i(3), wsplat])
            x2p = x2 + 1.0
            y2p = y2 + 1.0
            thr_area = thr * ((x2p - x1) * (y2p - y1))
            return (x1, y1, x2, y2, x2p, y2p, thr_area)

        def append(cand, slot):
            x1, y1, x2, y2, x2p, y2p, thr_area = cand
            csplat = _splat_i(slot)
            plsc.store_scatter(kept_v, [_splat_i(0), csplat], x1)
            plsc.store_scatter(kept_v, [_splat_i(1), csplat], y1)
            plsc.store_scatter(kept_v, [_splat_i(2), csplat], x2)
            plsc.store_scatter(kept_v, [_splat_i(3), csplat], y2)
            plsc.store_scatter(kx2p_v, [csplat], x2p)
            plsc.store_scatter(ky2p_v, [csplat], y2p)
            plsc.store_scatter(kathr_v, [csplat], thr_area)

        def cond(carry):
            count, picked = carry
            return (count < AFT_NMS_TOPK) & (picked < PRE_NMS_TOPK)

        def body(carry):
            count, picked = carry
            cands = [draw() for _ in range(4)]

            def iou_chunk(cols, accs):
                kx1 = plsc.load_gather(kept_v, [_splat_i(0), cols])
                ky1 = plsc.load_gather(kept_v, [_splat_i(1), cols])
                kx2p = plsc.load_gather(kx2p_v, [cols])
                ky2p = plsc.load_gather(ky2p_v, [cols])
                kathr = plsc.load_gather(kathr_v, [cols])
                out = []
                for (x1, y1, _, _, x2p, y2p, ta), acc in zip(cands, accs):
                    iw = jnp.maximum(0.0, jnp.minimum(x2p, kx2p) - jnp.maximum(x1, kx1))
                    ih = jnp.maximum(0.0, jnp.minimum(y2p, ky2p) - jnp.maximum(y1, ky1))
                    out.append(acc | (thr1 * (iw * ih) > ta + kathr))
                return tuple(out)

            def iou_pair(j, accs):
                cols = j * (2 * L) + iota
                return iou_chunk(cols + L, iou_chunk(cols, accs))

            npair = (count + (2 * L - 1)) // (2 * L)
            zero = jnp.zeros((L,), dtype=jnp.bool_)
            supp = lax.fori_loop(0, npair, iou_pair, (zero,) * 4)

            def pair_ov(ci, cj):
                x1i, y1i, _, _, x2pi, y2pi, tai = ci
                x1j, y1j, _, _, x2pj, y2pj, taj = cj
                iw = jnp.maximum(0.0, jnp.minimum(x2pi, x2pj) - jnp.maximum(x1i, x1j))
                ih = jnp.maximum(0.0, jnp.minimum(y2pi, y2pj) - jnp.maximum(y1i, y1j))
                return jnp.any(thr1 * (iw * ih) > tai + taj)

            keeps = []
            for i in range(4):
                k = jnp.logical_not(jnp.any(supp[i]))
                for j in range(i):
                    k = k & ~(keeps[j] & pair_ov(cands[j], cands[i]))
                keeps.append(k)

            slot = count
            for i in range(4):
                ki = keeps[i] & (slot < AFT_NMS_TOPK)
                s = slot

                @pl.when(ki)
                def _(ci=cands[i], s=s):
                    append(ci, s)

                slot = slot + ki.astype(jnp.int32)

            return (slot, picked + 4)

        count, _ = lax.while_loop(cond, body, (jnp.int32(0), jnp.int32(0)))

        k0 = [plsc.load_gather(kept_v, [_splat_i(r), _splat_i(0)])
              for r in range(4)]

        def pad_chunk(i, _):
            cols = i * L + iota
            live = cols < count
            for r in range(4):
                cur = plsc.load_gather(kept_v, [_splat_i(r), cols])
                plsc.store_scatter(kept_v, [_splat_i(r), cols],
                                   jnp.where(live, cur, k0[r]))
            return 0

        lax.fori_loop(0, KEEP_PAD // L, pad_chunk, 0)

        pltpu.sync_copy(kept_v, out_hbm)


_sc_nms = functools.partial(
    pl.kernel,
    out_type=jax.ShapeDtypeStruct((4, KEEP_PAD), jnp.float32),
    mesh=plsc.VectorSubcoreMesh(core_axis_name="c", subcore_axis_name="s"),
    compiler_params=pltpu.CompilerParams(needs_layout_passes=False),
    scratch_types=[
        pltpu.VMEM((NPAD,), jnp.float32),
        pltpu.VMEM((4, N), jnp.float32),
        pltpu.VMEM((NB1_PAD,), jnp.float32),
        pltpu.VMEM((NB2,), jnp.float32),
        pltpu.VMEM((4, KEEP_PAD), jnp.float32),
        pltpu.VMEM((KEEP_PAD,), jnp.float32),
        pltpu.VMEM((KEEP_PAD,), jnp.float32),
        pltpu.VMEM((KEEP_PAD,), jnp.float32),
        pltpu.SemaphoreType.DMA,
    ],
)(_sc_body)


def kernel(score, reg_param, anchors, im_info):
    score_fg = jnp.concatenate(
        [score[0, :, 0], jnp.full((NPAD - N,), NEG_INF, dtype=jnp.float32)])
    rois_t = _decode_rois(im_info.astype(jnp.float32), anchors.T, reg_param.T)
    out = _sc_nms(score_fg, rois_t)
    return out[:, :AFT_NMS_TOPK].T


# speedup vs baseline: 1.0977x; 1.0105x over previous
import functools

import jax
import jax.numpy as jnp
from jax import lax
from jax.experimental import pallas as pl
from jax.experimental.pallas import tpu as pltpu
from jax.experimental.pallas import tpu_sc as plsc

N = 20000
PRE_NMS_TOPK = 6000
AFT_NMS_TOPK = 300
RPN_NMS_THR = 0.7

L = 16
NPAD = 20224
NB1 = NPAD // L
NB1_PAD = 1280
NB2 = NB1_PAD // L
KEEP_PAD = 336
NEG_INF = float("-inf")


def _decode_body(im_ref, a_ref, r_ref, o_ref):
    a = a_ref[...]
    r = r_ref[...]
    x1, y1, x2, y2 = a[0:1], a[1:2], a[2:3], a[3:4]
    w = x2 - x1 + 1.0
    h = y2 - y1 + 1.0
    cx = x1 + 0.5 * w
    cy = y1 + 0.5 * h
    pcx = r[0:1] * w + cx
    pcy = r[1:2] * h + cy
    pw = jnp.exp(r[2:3]) * w
    ph = jnp.exp(r[3:4]) * h
    rois = jnp.concatenate(
        [pcx - 0.5 * pw, pcy - 0.5 * ph, pcx + 0.5 * pw, pcy + 0.5 * ph], axis=0)
    col = lax.broadcasted_iota(jnp.int32, rois.shape, 1)
    even = (col % 2) == 0
    lim0 = im_ref[0] - 1.0
    lim1 = im_ref[1] - 1.0
    o_ref[...] = jnp.where(even, jnp.clip(rois, 0.0, lim0),
                           jnp.clip(rois, 0.0, lim1))


def _decode_rois(im_info, anchors_t, reg_t):
    return pl.pallas_call(
        _decode_body,
        out_shape=jax.ShapeDtypeStruct((4, N), jnp.float32),
        in_specs=[
            pl.BlockSpec(memory_space=pltpu.SMEM),
            pl.BlockSpec(memory_space=pltpu.VMEM),
            pl.BlockSpec(memory_space=pltpu.VMEM),
        ],
        out_specs=pl.BlockSpec(memory_space=pltpu.VMEM),
    )(im_info, anchors_t, reg_t)


def _splat_f(x):
    return jnp.full((L,), x, dtype=jnp.float32)


def _splat_i(x):
    return jnp.full((L,), x, dtype=jnp.int32)


def _sc_body(score_hbm, rois_hbm, out_hbm, sc_v, rois_v, bm1_v, bm2_v,
             kept_v, kx2p_v, ky2p_v, kathr_v, sem):
    iota = lax.iota(jnp.int32, L)
    big = jnp.int32(0x7FFFFFF)
    thr = jnp.float32(RPN_NMS_THR)
    thr1 = jnp.float32(1.0 + RPN_NMS_THR)

    @pl.when((lax.axis_index("c") == 0) & (lax.axis_index("s") == 0))
    def _tile0():
        rois_cp = pltpu.async_copy(rois_hbm, rois_v, sem)
        pltpu.sync_copy(score_hbm, sc_v)

        def bm1_chunk(k, _):
            m = _splat_f(NEG_INF)

            def inner(j, m):
                v = plsc.load_gather(sc_v, [(k * 256 + iota * L) + j])
                return jnp.maximum(m, v)

            m = lax.fori_loop(0, L, inner, m, unroll=4)
            plsc.store_scatter(bm1_v, [k * L + iota], m)
            return 0

        lax.fori_loop(0, NB1 // L, bm1_chunk, 0)
        bm1_v[pl.ds(NB1, L)] = _splat_f(NEG_INF)

        def bm2_one(t, _):
            v = plsc.load_gather(bm1_v, [t * L + iota])
            plsc.store_scatter(bm2_v, [_splat_i(t)], _splat_f(jnp.max(v)))
            return 0

        lax.fori_loop(0, NB2, bm2_one, 0)

        def kept_init(i, _):
            cols = i * L + iota
            plsc.store_scatter(kept_v, [_splat_i(0), cols], _splat_f(3e9))
            plsc.store_scatter(kept_v, [_splat_i(1), cols], _splat_f(3e9))
            plsc.store_scatter(kept_v, [_splat_i(2), cols], _splat_f(-3e9))
            plsc.store_scatter(kept_v, [_splat_i(3), cols], _splat_f(-3e9))
            plsc.store_scatter(kx2p_v, [cols], _splat_f(-3e9))
            plsc.store_scatter(ky2p_v, [cols], _splat_f(-3e9))
            plsc.store_scatter(kathr_v, [cols], _splat_f(1.0))
            return 0

        lax.fori_loop(0, KEEP_PAD // L, kept_init, 0)

        rois_cp.wait()

        def first_min_idx(vals, maxv, idxs):
            return jnp.min(jnp.where(vals == maxv, idxs, big))

        def draw():
            vs = [(bm2_v[pl.ds(t * L, L)], t * L + iota)
                  for t in range(NB2 // L)]
            while len(vs) > 1:
                nxt = []
                for i in range(0, len(vs) - 1, 2):
                    (va, ka), (vb, kb) = vs[i], vs[i + 1]
                    later = vb > va
                    nxt.append((jnp.where(later, vb, va),
                                jnp.where(later, kb, ka)))
                if len(vs) % 2:
                    nxt.append(vs[-1])
                vs = nxt
            bestv, bestk = vs[0]
            m2 = jnp.max(bestv)
            k_star = first_min_idx(bestv, m2, bestk)
            v1 = plsc.load_gather(bm1_v, [k_star * L + iota])
            l1 = plsc.all_reduce_ffs(v1 == jnp.max(v1))
            b_star = k_star * L + l1
            v0 = plsc.load_gather(sc_v, [b_star * L + iota])
            l0 = plsc.all_reduce_ffs(v0 == jnp.max(v0))
            widx = b_star * L + l0

            plsc.store_scatter(sc_v, [_splat_i(widx)], _splat_f(NEG_INF))
            v0n = jnp.where(iota == l0, NEG_INF, v0)
            bm1_new = jnp.max(v0n)
            plsc.store_scatter(bm1_v, [_splat_i(b_star)], _splat_f(bm1_new))
            v1n = jnp.where(iota == l1, bm1_new, v1)
            plsc.store_scatter(bm2_v, [_splat_i(k_star)], _splat_f(jnp.max(v1n)))

            wsplat = _splat_i(widx)
            x1 = plsc.load_gather(rois_v, [_splat_i(0), wsplat])
            y1 = plsc.load_gather(rois_v, [_splat_i(1), wsplat])
            x2 = plsc.load_gather(rois_v, [_splat_i(2), wsplat])
            y2 = plsc.load_gather(rois_v, [_splat_i(3), wsplat])
            x2p = x2 + 1.0
            y2p = y2 + 1.0
            thr_area = thr * ((x2p - x1) * (y2p - y1))
            return (x1, y1, x2, y2, x2p, y2p, thr_area)

        def append(cand, slot):
            x1, y1, x2, y2, x2p, y2p, thr_area = cand
            csplat = _splat_i(slot)
            plsc.store_scatter(kept_v, [_splat_i(0), csplat], x1)
            plsc.store_scatter(kept_v, [_splat_i(1), csplat], y1)
            plsc.store_scatter(kept_v, [_splat_i(2), csplat], x2)
            plsc.store_scatter(kept_v, [_splat_i(3), csplat], y2)
            plsc.store_scatter(kx2p_v, [csplat], x2p)
            plsc.store_scatter(ky2p_v, [csplat], y2p)
            plsc.store_scatter(kathr_v, [csplat], thr_area)

        def cond(carry):
            count, picked = carry
            return (count < AFT_NMS_TOPK) & (picked < PRE_NMS_TOPK)

        def body(carry):
            count, picked = carry
            cands = [draw() for _ in range(4)]

            def iou_chunk(cols, accs):
                kx1 = plsc.load_gather(kept_v, [_splat_i(0), cols])
                ky1 = plsc.load_gather(kept_v, [_splat_i(1), cols])
                kx2p = plsc.load_gather(kx2p_v, [cols])
                ky2p = plsc.load_gather(ky2p_v, [cols])
                kathr = plsc.load_gather(kathr_v, [cols])
                out = []
                for (x1, y1, _, _, x2p, y2p, ta), acc in zip(cands, accs):
                    iw = jnp.maximum(0.0, jnp.minimum(x2p, kx2p) - jnp.maximum(x1, kx1))
                    ih = jnp.maximum(0.0, jnp.minimum(y2p, ky2p) - jnp.maximum(y1, ky1))
                    out.append(acc | (thr1 * (iw * ih) > ta + kathr))
                return tuple(out)

            def iou_one(j, accs):
                return iou_chunk(j * L + iota, accs)

            nk = (count + (L - 1)) // L
            zero = jnp.zeros((L,), dtype=jnp.bool_)
            supp = lax.fori_loop(0, nk, iou_one, (zero,) * 4)

            def pair_ov(ci, cj):
                x1i, y1i, _, _, x2pi, y2pi, tai = ci
                x1j, y1j, _, _, x2pj, y2pj, taj = cj
                iw = jnp.maximum(0.0, jnp.minimum(x2pi, x2pj) - jnp.maximum(x1i, x1j))
                ih = jnp.maximum(0.0, jnp.minimum(y2pi, y2pj) - jnp.maximum(y1i, y1j))
                return jnp.any(thr1 * (iw * ih) > tai + taj)

            keeps = []
            for i in range(4):
                k = jnp.logical_not(jnp.any(supp[i]))
                for j in range(i):
                    k = k & ~(keeps[j] & pair_ov(cands[j], cands[i]))
                keeps.append(k)

            slot = count
            for i in range(4):
                ki = keeps[i] & (slot < AFT_NMS_TOPK)
                s = slot

                @pl.when(ki)
                def _(ci=cands[i], s=s):
                    append(ci, s)

                slot = slot + ki.astype(jnp.int32)

            return (slot, picked + 4)

        count, _ = lax.while_loop(cond, body, (jnp.int32(0), jnp.int32(0)))

        k0 = [plsc.load_gather(kept_v, [_splat_i(r), _splat_i(0)])
              for r in range(4)]

        def pad_chunk(i, _):
            cols = i * L + iota
            live = cols < count
            for r in range(4):
                cur = plsc.load_gather(kept_v, [_splat_i(r), cols])
                plsc.store_scatter(kept_v, [_splat_i(r), cols],
                                   jnp.where(live, cur, k0[r]))
            return 0

        lax.fori_loop(0, KEEP_PAD // L, pad_chunk, 0)

        pltpu.sync_copy(kept_v, out_hbm)


_sc_nms = functools.partial(
    pl.kernel,
    out_type=jax.ShapeDtypeStruct((4, KEEP_PAD), jnp.float32),
    mesh=plsc.VectorSubcoreMesh(core_axis_name="c", subcore_axis_name="s"),
    compiler_params=pltpu.CompilerParams(needs_layout_passes=False),
    scratch_types=[
        pltpu.VMEM((NPAD,), jnp.float32),
        pltpu.VMEM((4, N), jnp.float32),
        pltpu.VMEM((NB1_PAD,), jnp.float32),
        pltpu.VMEM((NB2,), jnp.float32),
        pltpu.VMEM((4, KEEP_PAD), jnp.float32),
        pltpu.VMEM((KEEP_PAD,), jnp.float32),
        pltpu.VMEM((KEEP_PAD,), jnp.float32),
        pltpu.VMEM((KEEP_PAD,), jnp.float32),
        pltpu.SemaphoreType.DMA,
    ],
)(_sc_body)


def kernel(score, reg_param, anchors, im_info):
    score_fg = jnp.concatenate(
        [score[0, :, 0], jnp.full((NPAD - N,), NEG_INF, dtype=jnp.float32)])
    rois_t = _decode_rois(im_info.astype(jnp.float32), anchors.T, reg_param.T)
    out = _sc_nms(score_fg, rois_t)
    return out[:, :AFT_NMS_TOPK].T
